# baseline (speedup 1.0000x reference)
import functools

import jax
import jax.numpy as jnp
from jax import lax
from jax.experimental import pallas as pl
from jax.experimental.pallas import tpu as pltpu
from jax.experimental.pallas import tpu_sc as plsc

_B = 16
_P = 512
_D = 512
_F = 2048
_HALF = _F // 2
_CHUNK = 64
_NBUF = 3
_NCHUNKS = _HALF // _CHUNK


@functools.partial(
    pl.kernel,
    out_type=[
        jax.ShapeDtypeStruct((_B * _F, _D), jnp.float32),
        jax.ShapeDtypeStruct((_B * _F,), jnp.int32),
    ],
    mesh=plsc.VectorSubcoreMesh(core_axis_name="c", subcore_axis_name="s"),
    compiler_params=pltpu.CompilerParams(needs_layout_passes=False),
    scratch_types=[
        pltpu.VMEM((_P,), jnp.int32),
        pltpu.VMEM((_F,), jnp.int32),
        pltpu.VMEM((_F // _CHUNK, _CHUNK), jnp.int32),
        pltpu.VMEM((_F,), jnp.int32),
        pltpu.VMEM((_NBUF, _CHUNK, _D), jnp.float32),
        pltpu.SemaphoreType.DMA,
        pltpu.SemaphoreType.DMA,
        pltpu.SemaphoreType.DMA,
    ],
)
def _length_regulate(x_hbm, dur_hbm, out_hbm, val_hbm,
                     dur_v, sbuf, idx2, val_v, rows, sem0, sem1, sem2):
    sems = (sem0, sem1, sem2)
    b = lax.axis_index("s")
    half = lax.axis_index("c")

    pltpu.sync_copy(dur_hbm.at[b], dur_v)

    zero = jnp.zeros((16,), jnp.int32)
    for j in range(_F // 16):
        sbuf[pl.ds(j * 16, 16)] = zero

    one = jnp.ones((16,), jnp.int32)
    carry = jnp.int32(0)
    for j in range(_P // 16):
        v = dur_v[pl.ds(j * 16, 16)]
        cum = plsc.cumsum(v) + carry
        carry = carry + jnp.sum(v)
        plsc.store_scatter(sbuf, [cum], one, mask=cum < _F)

    base = b * _P
    carry = jnp.int32(0)
    for j in range(_F // 16):
        v = sbuf[pl.ds(j * 16, 16)]
        s = plsc.cumsum(v) + carry
        carry = carry + jnp.sum(v)
        val_v[pl.ds(j * 16, 16)] = s
        gidx = jnp.minimum(s + base, _B * _P - 1)
        idx2[j // (_CHUNK // 16), pl.ds((j % (_CHUNK // 16)) * 16, 16)] = gidx

    row0 = b * _F + half * _HALF
    pltpu.sync_copy(val_v.at[pl.ds(half * _HALF, _HALF)],
                    val_hbm.at[pl.ds(row0, _HALF)])

    crow0 = half * _NCHUNKS
    handles = {}
    for c in range(_NBUF):
        handles[("g", c)] = pltpu.async_copy(
            x_hbm.at[idx2.at[crow0 + c]], rows.at[c], sems[c])
    for c in range(_NCHUNKS):
        bb = c % _NBUF
        handles[("g", c)].wait()
        h = pltpu.async_copy(
            rows.at[bb], out_hbm.at[pl.ds(row0 + c * _CHUNK, _CHUNK)],
            sems[bb])
        handles[("o", c)] = h
        if c + _NBUF < _NCHUNKS:
            h.wait()
            handles[("g", c + _NBUF)] = pltpu.async_copy(
                x_hbm.at[idx2.at[crow0 + c + _NBUF]], rows.at[bb], sems[bb])
    for c in range(_NCHUNKS - _NBUF, _NCHUNKS):
        handles[("o", c)].wait()


def kernel(x, durations):
    B, P, D = x.shape
    x_flat = x.reshape(B * P, D)
    out_flat, val = _length_regulate(x_flat, durations)
    out = out_flat.reshape(B, _F, D)
    val = val.reshape(B, _F)
    return out, val == (P - 1)


# speedup vs baseline: 1.0274x; 1.0274x over previous
import functools

import jax
import jax.numpy as jnp
from jax import lax
from jax.experimental import pallas as pl
from jax.experimental.pallas import tpu as pltpu
from jax.experimental.pallas import tpu_sc as plsc

_B = 16
_P = 512
_D = 512
_F = 2048
_HALF = _F // 2
_CHUNK = 64
_NBUF = 3
_NCHUNKS = _HALF // _CHUNK


@functools.partial(
    pl.kernel,
    out_type=[
        jax.ShapeDtypeStruct((_B * _F, _D), jnp.float32),
        jax.ShapeDtypeStruct((_B * _F,), jnp.int32),
    ],
    mesh=plsc.VectorSubcoreMesh(core_axis_name="c", subcore_axis_name="s"),
    compiler_params=pltpu.CompilerParams(needs_layout_passes=False),
    scratch_types=[
        pltpu.VMEM((_P,), jnp.int32),
        pltpu.VMEM((_F,), jnp.int32),
        pltpu.VMEM((_F // _CHUNK, _CHUNK), jnp.int32),
        pltpu.VMEM((_F,), jnp.int32),
        pltpu.VMEM((_NBUF, _CHUNK, _D), jnp.float32),
        pltpu.SemaphoreType.DMA,
        pltpu.SemaphoreType.DMA,
        pltpu.SemaphoreType.DMA,
    ],
)
def _length_regulate(x_hbm, dur_hbm, out_hbm, val_hbm,
                     dur_v, sbuf, idx2, val_v, rows, sem0, sem1, sem2):
    sems = (sem0, sem1, sem2)
    b = lax.axis_index("s")
    half = lax.axis_index("c")

    pltpu.sync_copy(dur_hbm.at[b], dur_v)

    zero = jnp.zeros((16,), jnp.int32)
    for j in range(_F // 16):
        sbuf[pl.ds(j * 16, 16)] = zero

    one = jnp.ones((16,), jnp.int32)
    carry = jnp.int32(0)
    for j in range(_P // 16):
        v = dur_v[pl.ds(j * 16, 16)]
        cum = plsc.cumsum(v) + carry
        carry = carry + jnp.sum(v)
        plsc.store_scatter(sbuf, [cum], one, mask=cum < _F)

    base = b * _P
    carry = jnp.int32(0)
    for j in range(_F // 16):
        v = sbuf[pl.ds(j * 16, 16)]
        s = plsc.cumsum(v) + carry
        carry = carry + jnp.sum(v)
        val_v[pl.ds(j * 16, 16)] = s
        gidx = jnp.minimum(s + base, _B * _P - 1)
        idx2[j // (_CHUNK // 16), pl.ds((j % (_CHUNK // 16)) * 16, 16)] = gidx

    @pl.when(half == 0)
    def _():
        pltpu.sync_copy(val_v, val_hbm.at[pl.ds(b * _F, _F)])

    handles = {}

    def chunk_row(c):
        return pl.ds(b * _F + (2 * c + half) * _CHUNK, _CHUNK)

    def idx_row(c):
        return 2 * c + half

    for c in range(_NBUF):
        handles[("g", c)] = pltpu.async_copy(
            x_hbm.at[idx2.at[idx_row(c)]], rows.at[c], sems[c])
    for c in range(_NCHUNKS):
        bb = c % _NBUF
        handles[("g", c)].wait()
        h = pltpu.async_copy(rows.at[bb], out_hbm.at[chunk_row(c)], sems[bb])
        handles[("o", c)] = h
        if c + _NBUF < _NCHUNKS:
            h.wait()
            handles[("g", c + _NBUF)] = pltpu.async_copy(
                x_hbm.at[idx2.at[idx_row(c + _NBUF)]], rows.at[bb], sems[bb])
    for c in range(_NCHUNKS - _NBUF, _NCHUNKS):
        handles[("o", c)].wait()


def kernel(x, durations):
    B, P, D = x.shape
    x_flat = x.reshape(B * P, D)
    out_flat, val = _length_regulate(x_flat, durations)
    out = out_flat.reshape(B, _F, D)
    val = val.reshape(B, _F)
    return out, val == (P - 1)
